# baseline (speedup 1.0000x reference)
import dataclasses
import functools

import jax
import jax.numpy as jnp
from jax.experimental import pallas as pl
from jax.experimental.pallas import tpu as pltpu
from jax.experimental.pallas import tpu_sc as plsc

N = 4096
D = 128
D_OUT = 64
K_IDX = 16609443

BM = 512
NB = N // BM

NW = 32
LANES = 16
H1_ROWS = 130
H2_ROWS = 128

_SC_CP = pltpu.CompilerParams()
if "needs_layout_passes" in pltpu.CompilerParams.__dataclass_fields__:
    _SC_CP = dataclasses.replace(_SC_CP, needs_layout_passes=False)


def _prep_kernel(x_ref, w_in_ref, xn_ref, h_ref):
    x = x_ref[...]
    nrm = jnp.sqrt(jnp.sum(x * x, axis=1, keepdims=True))
    xn_ref[...] = x / jnp.maximum(nrm, 1e-8)
    h = jax.lax.dot_general(
        x, w_in_ref[...], (((1,), (1,)), ((), ())),
        preferred_element_type=jnp.float32,
    )
    h_ref[...] = jnp.maximum(h, 0.0)


def _sim_kernel(xn_blk_ref, xn_ref, bits_ref):
    s = jax.lax.dot_general(
        xn_blk_ref[...], xn_ref[...], (((1,), (1,)), ((), ())),
        preferred_element_type=jnp.float32,
    )
    bits_ref[...] = pltpu.bitcast(jnp.abs(s), jnp.int32)


SC_BLK = 16384
SC_UNROLL = 8


def _make_sc_hist(shift, binmask, match_shift, nbins):
    def sc_hist(bits_hbm, sel_hbm, out_hbm, hist_v, sel_v, sem):
        wid = jax.lax.axis_index("s") * 2 + jax.lax.axis_index("c")
        pltpu.async_copy(sel_hbm, sel_v, sem).wait()
        zeros = jnp.zeros((LANES,), jnp.int32)

        @pl.loop(0, nbins * LANES, step=16 * LANES)
        def _zero(i):
            for u in range(16):
                hist_v.at[pl.ds(i + u * LANES, LANES)][...] = zeros

        ones = jnp.ones((LANES,), jnp.int32)
        lane = jax.lax.iota(jnp.int32, LANES)
        bm = jnp.full((LANES,), binmask, jnp.int32)
        pvec = sel_v.at[0][...]

        def body(in_v):
            @pl.loop(0, SC_BLK, step=SC_UNROLL * LANES)
            def _(c):
                for u in range(SC_UNROLL):
                    v = in_v.at[pl.ds(c + u * LANES, LANES)][...]
                    hi = jax.lax.shift_right_logical(v, match_shift)
                    b = jnp.bitwise_and(
                        jax.lax.shift_right_logical(v, shift), bm)
                    idx = jax.lax.shift_left(b, 4) + lane
                    plsc.addupdate_scatter(hist_v, [idx], ones,
                                           mask=hi == pvec)

        pltpu.emit_pipeline(
            body,
            grid=(N * N // SC_BLK,),
            in_specs=[pl.BlockSpec((SC_BLK,), lambda i: (i,))],
            out_specs=[],
            core_axis_name=("c", "s"),
            dimension_semantics=(pltpu.PARALLEL,),
        )(bits_hbm)
        pltpu.async_copy(hist_v, out_hbm.at[wid], sem).wait()

    return sc_hist


def _rowmajor_cum(hs, rows, cols):
    ric = jax.lax.broadcasted_iota(jnp.int32, (cols, cols), 0)
    cic = jax.lax.broadcasted_iota(jnp.int32, (cols, cols), 1)
    ut = (ric <= cic).astype(jnp.float32)
    cum_row = jax.lax.dot_general(
        hs, ut, (((1,), (0,)), ((), ())), preferred_element_type=jnp.float32)
    tot = cum_row[:, cols - 1:cols]
    rir = jax.lax.broadcasted_iota(jnp.int32, (rows, rows), 0)
    cir = jax.lax.broadcasted_iota(jnp.int32, (rows, rows), 1)
    lt = (cir < rir).astype(jnp.float32)
    prev = jax.lax.dot_general(
        lt, tot, (((1,), (0,)), ((), ())), preferred_element_type=jnp.float32)
    return cum_row + prev


def _make_pick(rows, shift):
    def pick(h_ref, sel_ref, out_ref):
        hs = jnp.sum(h_ref[...].astype(jnp.float32), axis=0)
        gic = jax.lax.broadcasted_iota(jnp.int32, (128, 8), 0)
        ggc = jax.lax.broadcasted_iota(jnp.int32, (128, 8), 1)
        red = (jax.lax.shift_right_logical(gic, 4) == ggc).astype(jnp.float32)
        hg = jax.lax.dot_general(
            hs, red, (((1,), (0,)), ((), ())),
            preferred_element_type=jnp.float32)
        cum = _rowmajor_cum(hg, rows, 8)
        prev_rank = sel_ref[1, 0]
        r1 = prev_rank.astype(jnp.float32) + 1.0
        lin = (jax.lax.broadcasted_iota(jnp.int32, (rows, 8), 0) * 8
               + jax.lax.broadcasted_iota(jnp.int32, (rows, 8), 1))
        mask = cum >= r1
        bsel = jnp.min(jnp.where(mask, lin, jnp.int32(2 ** 30)))
        cum_at_b = jnp.min(jnp.where(mask, cum, jnp.float32(3e8)))
        hist_at_b = jnp.sum(jnp.where(lin == bsel, hg, 0.0))
        count_below = (cum_at_b - hist_at_b).astype(jnp.int32)
        new_rank = prev_rank - count_below
        new_prefix = jnp.left_shift(sel_ref[0, 0], shift) | bsel
        rowi = jax.lax.broadcasted_iota(jnp.int32, (2, LANES), 0)
        out_ref[...] = jnp.where(rowi == 0, new_prefix, new_rank)

    return pick


def _sage_kernel(eps_ref, bits_ref, h_ref, h_blk_ref, wl_ref, bl_ref,
                 wr_ref, wo_ref, bo_ref, out_ref):
    eps = eps_ref[0]
    mask = (bits_ref[...] >= eps).astype(jnp.float32)
    deg = jnp.sum(mask, axis=1, keepdims=True)
    aggn = jnp.dot(mask, h_ref[...], preferred_element_type=jnp.float32)
    agg = aggn / jnp.maximum(deg, 1.0)
    z = (
        jax.lax.dot_general(
            agg, wl_ref[...], (((1,), (1,)), ((), ())),
            preferred_element_type=jnp.float32,
        )
        + bl_ref[...]
        + jax.lax.dot_general(
            h_blk_ref[...], wr_ref[...], (((1,), (1,)), ((), ())),
            preferred_element_type=jnp.float32,
        )
    )
    h2 = jnp.maximum(z, 0.0)
    o = jax.lax.dot_general(
        h2, wo_ref[...], (((1,), (1,)), ((), ())),
        preferred_element_type=jnp.float32,
    ) + bo_ref[...]
    out_ref[...] = jax.nn.sigmoid(o)


@jax.jit
def kernel(x, W_in, W_l, b_l, W_r, W_out, b_out):
    xn, h = pl.pallas_call(
        _prep_kernel,
        out_shape=(
            jax.ShapeDtypeStruct((N, D), jnp.float32),
            jax.ShapeDtypeStruct((N, D), jnp.float32),
        ),
    )(x, W_in)

    bits = pl.pallas_call(
        _sim_kernel,
        grid=(NB,),
        in_specs=[
            pl.BlockSpec((BM, D), lambda i: (i, 0)),
            pl.BlockSpec((N, D), lambda i: (0, 0)),
        ],
        out_specs=pl.BlockSpec((BM, N), lambda i: (i, 0)),
        out_shape=jax.ShapeDtypeStruct((N, N), jnp.int32),
    )(xn, xn)

    mesh = plsc.VectorSubcoreMesh(core_axis_name="c", subcore_axis_name="s")
    bits_flat = bits.reshape(-1)

    def sc_hist(sel, shift, binmask, match_shift, nbins):
        return functools.partial(
            pl.kernel,
            out_type=jax.ShapeDtypeStruct((NW, nbins * LANES), jnp.int32),
            mesh=mesh,
            compiler_params=_SC_CP,
            scratch_types=[
                pltpu.VMEM((nbins * LANES,), jnp.int32),
                pltpu.VMEM((2, LANES), jnp.int32),
                pltpu.SemaphoreType.DMA,
            ],
        )(_make_sc_hist(shift, binmask, match_shift, nbins))(bits_flat, sel)

    def pick(h, sel, rows, shift):
        return pl.pallas_call(
            _make_pick(rows, shift),
            in_specs=[
                pl.BlockSpec((NW, rows, 128), lambda: (0, 0, 0)),
                pl.BlockSpec(memory_space=pltpu.SMEM),
            ],
            out_shape=jax.ShapeDtypeStruct((2, LANES), jnp.int32),
        )(h.reshape(NW, rows, 128), sel)

    sel0 = jnp.concatenate([
        jnp.zeros((1, LANES), jnp.int32),
        jnp.full((1, LANES), K_IDX, jnp.int32),
    ])
    h1 = sc_hist(sel0, 20, 2047, 31, H1_ROWS * 8)
    sel1 = pick(h1, sel0, H1_ROWS, 11)
    h2 = sc_hist(sel1, 10, 1023, 20, H2_ROWS * 8)
    sel2 = pick(h2, sel1, H2_ROWS, 10)
    h3 = sc_hist(sel2, 0, 1023, 10, H2_ROWS * 8)
    sel3 = pick(h3, sel2, H2_ROWS, 10)
    eps_bits = jax.lax.slice(sel3, (0, 0), (1, 1))

    out = pl.pallas_call(
        _sage_kernel,
        grid=(NB,),
        in_specs=[
            pl.BlockSpec(memory_space=pltpu.SMEM),
            pl.BlockSpec((BM, N), lambda i: (i, 0)),
            pl.BlockSpec((N, D), lambda i: (0, 0)),
            pl.BlockSpec((BM, D), lambda i: (i, 0)),
            pl.BlockSpec((D, D), lambda i: (0, 0)),
            pl.BlockSpec((1, D), lambda i: (0, 0)),
            pl.BlockSpec((D, D), lambda i: (0, 0)),
            pl.BlockSpec((D_OUT, D), lambda i: (0, 0)),
            pl.BlockSpec((1, D_OUT), lambda i: (0, 0)),
        ],
        out_specs=pl.BlockSpec((BM, D_OUT), lambda i: (i, 0)),
        out_shape=jax.ShapeDtypeStruct((N, D_OUT), jnp.float32),
    )(
        eps_bits.reshape(-1), bits, h, h,
        W_l, b_l.reshape(1, D), W_r, W_out, b_out.reshape(1, D_OUT),
    )
    return out


# speedup vs baseline: 2.1382x; 2.1382x over previous
import jax
import jax.numpy as jnp
from jax.experimental import pallas as pl
from jax.experimental.pallas import tpu as pltpu

N = 4096
D = 128
D_OUT = 64
K_IDX = 16609443
HI_BITS = 0x40000000
N_PASSES = 16

BM = 1024
NB = N // BM


def _prep_kernel(x_ref, w_in_ref, xn_ref, h_ref):
    x = x_ref[...]
    nrm = jnp.sqrt(jnp.sum(x * x, axis=1, keepdims=True))
    xn_ref[...] = x / jnp.maximum(nrm, 1e-8)
    h = jax.lax.dot_general(
        x, w_in_ref[...], (((1,), (1,)), ((), ())),
        preferred_element_type=jnp.float32,
    )
    h_ref[...] = jnp.maximum(h, 0.0)


def _sim_kernel(xn_blk_ref, xn_ref, bits_ref):
    s = jax.lax.dot_general(
        xn_blk_ref[...], xn_ref[...], (((1,), (1,)), ((), ())),
        preferred_element_type=jnp.float32,
    )
    bits_ref[...] = pltpu.bitcast(jnp.abs(s), jnp.int32)


def _select_kernel(bits_ref, eps_ref, state_ref, acc_ref):
    p = pl.program_id(0)
    b = pl.program_id(1)
    K1 = float(K_IDX + 1)

    def _thresholds(lo, hi):
        t2 = lo + (hi - lo) // 2
        t1 = lo + (t2 - lo) // 2
        t3 = t2 + (hi - t2) // 2
        return t1, t2, t3

    def _narrow(lo, hi):
        t1, t2, t3 = _thresholds(lo, hi)
        c1 = acc_ref[0]
        c2 = acc_ref[1]
        c3 = acc_ref[2]
        new_hi = jnp.where(c1 >= K1, t1,
                  jnp.where(c2 >= K1, t2,
                   jnp.where(c3 >= K1, t3, hi)))
        new_lo = jnp.where(c1 >= K1, lo,
                  jnp.where(c2 >= K1, t1,
                   jnp.where(c3 >= K1, t2, t3)))
        return new_lo, new_hi

    @pl.when(jnp.logical_and(p == 0, b == 0))
    def _init():
        state_ref[0] = -1
        state_ref[1] = HI_BITS
        acc_ref[0] = 0.0
        acc_ref[1] = 0.0
        acc_ref[2] = 0.0

    @pl.when(jnp.logical_and(p > 0, b == 0))
    def _update():
        new_lo, new_hi = _narrow(state_ref[0], state_ref[1])
        state_ref[0] = new_lo
        state_ref[1] = new_hi
        acc_ref[0] = 0.0
        acc_ref[1] = 0.0
        acc_ref[2] = 0.0

    t1, t2, t3 = _thresholds(state_ref[0], state_ref[1])
    blk = bits_ref[...]
    acc_ref[0] = acc_ref[0] + jnp.sum((blk <= t1).astype(jnp.float32))
    acc_ref[1] = acc_ref[1] + jnp.sum((blk <= t2).astype(jnp.float32))
    acc_ref[2] = acc_ref[2] + jnp.sum((blk <= t3).astype(jnp.float32))

    @pl.when(jnp.logical_and(p == N_PASSES - 1, b == NB - 1))
    def _final():
        _, new_hi = _narrow(state_ref[0], state_ref[1])
        eps_ref[0, 0] = new_hi


def _sage_kernel(eps_ref, bits_ref, h_ref, h_blk_ref, wl_ref, bl_ref,
                 wr_ref, wo_ref, bo_ref, out_ref):
    eps = eps_ref[0]
    mask = (bits_ref[...] >= eps).astype(jnp.float32)
    deg = jnp.sum(mask, axis=1, keepdims=True)
    aggn = jnp.dot(mask, h_ref[...], preferred_element_type=jnp.float32)
    agg = aggn / jnp.maximum(deg, 1.0)
    z = (
        jax.lax.dot_general(
            agg, wl_ref[...], (((1,), (1,)), ((), ())),
            preferred_element_type=jnp.float32,
        )
        + bl_ref[...]
        + jax.lax.dot_general(
            h_blk_ref[...], wr_ref[...], (((1,), (1,)), ((), ())),
            preferred_element_type=jnp.float32,
        )
    )
    h2 = jnp.maximum(z, 0.0)
    o = jax.lax.dot_general(
        h2, wo_ref[...], (((1,), (1,)), ((), ())),
        preferred_element_type=jnp.float32,
    ) + bo_ref[...]
    out_ref[...] = jax.nn.sigmoid(o)


@jax.jit
def kernel(x, W_in, W_l, b_l, W_r, W_out, b_out):
    xn, h = pl.pallas_call(
        _prep_kernel,
        out_shape=(
            jax.ShapeDtypeStruct((N, D), jnp.float32),
            jax.ShapeDtypeStruct((N, D), jnp.float32),
        ),
    )(x, W_in)

    bits = pl.pallas_call(
        _sim_kernel,
        grid=(NB,),
        in_specs=[
            pl.BlockSpec((BM, D), lambda i: (i, 0)),
            pl.BlockSpec((N, D), lambda i: (0, 0)),
        ],
        out_specs=pl.BlockSpec((BM, N), lambda i: (i, 0)),
        out_shape=jax.ShapeDtypeStruct((N, N), jnp.int32),
    )(xn, xn)

    eps_bits = pl.pallas_call(
        _select_kernel,
        grid=(N_PASSES, NB),
        in_specs=[pl.BlockSpec((BM, N), lambda p, b: (b, 0))],
        out_specs=pl.BlockSpec(memory_space=pltpu.SMEM),
        out_shape=jax.ShapeDtypeStruct((1, 1), jnp.int32),
        scratch_shapes=[
            pltpu.SMEM((2,), jnp.int32),
            pltpu.SMEM((3,), jnp.float32),
        ],
    )(bits)

    out = pl.pallas_call(
        _sage_kernel,
        grid=(NB,),
        in_specs=[
            pl.BlockSpec(memory_space=pltpu.SMEM),
            pl.BlockSpec((BM, N), lambda i: (i, 0)),
            pl.BlockSpec((N, D), lambda i: (0, 0)),
            pl.BlockSpec((BM, D), lambda i: (i, 0)),
            pl.BlockSpec((D, D), lambda i: (0, 0)),
            pl.BlockSpec((1, D), lambda i: (0, 0)),
            pl.BlockSpec((D, D), lambda i: (0, 0)),
            pl.BlockSpec((D_OUT, D), lambda i: (0, 0)),
            pl.BlockSpec((1, D_OUT), lambda i: (0, 0)),
        ],
        out_specs=pl.BlockSpec((BM, D_OUT), lambda i: (i, 0)),
        out_shape=jax.ShapeDtypeStruct((N, D_OUT), jnp.float32),
    )(
        eps_bits.reshape(-1), bits, h, h,
        W_l, b_l.reshape(1, D), W_r, W_out, b_out.reshape(1, D_OUT),
    )
    return out
